# baseline (device time: 159371 ns/iter reference)
import jax
import jax.numpy as jnp
from jax import lax
from jax.experimental import pallas as pl
from jax.experimental.pallas import tpu as pltpu

N_DEV = 16
M_BLK = 512
N_OUT = 4096
N_CHUNK = 4
N_SEG = N_OUT // N_CHUNK

ALT = [0]
for _d in range(1, 9):
    ALT.append(_d)
    if _d != 8:
        ALT.append(16 - _d)
SEND_ORDER = ALT[1:][::-1]


def _rel_of(tt):
    return jnp.where(tt == 0, 0, jnp.where(tt % 2 == 1, (tt + 1) // 2, 16 - tt // 2))


def kernel(x, w_mat):
    m_glob, k_shard = x.shape
    assert m_glob == N_DEV * M_BLK and k_shard == M_BLK

    def body(
        x_ref, w_hbm, out_ref, send_buf, comm_ref, w_buf, send_sems, recv_sems, w_sems
    ):
        t = pl.program_id(0)
        my = lax.axis_index("i")
        s_cur = (my + _rel_of(t)) % N_DEV
        s_nxt = (my + _rel_of(t + 1)) % N_DEV

        @pl.when(t == 0)
        def _setup():
            pltpu.make_async_copy(
                w_hbm.at[pl.ds(s_cur * M_BLK, M_BLK), :], w_buf.at[0], w_sems.at[0]
            ).start()

            barrier = pltpu.get_barrier_semaphore()
            for d in range(N_DEV):
                pl.semaphore_signal(
                    barrier,
                    inc=1,
                    device_id=(d,),
                    device_id_type=pl.DeviceIdType.MESH,
                )
            pl.semaphore_wait(barrier, N_DEV)

            for k, r in enumerate(SEND_ORDER):
                tgt = (my + r) % N_DEV
                send_buf[tgt] = x_ref[pl.ds(tgt * M_BLK, M_BLK), :].astype(
                    jnp.bfloat16
                )
                rdma = pltpu.make_async_remote_copy(
                    src_ref=send_buf.at[tgt],
                    dst_ref=comm_ref.at[my],
                    send_sem=send_sems.at[k],
                    recv_sem=recv_sems.at[my],
                    device_id=(tgt,),
                    device_id_type=pl.DeviceIdType.MESH,
                )
                rdma.start()

            comm_ref[my] = x_ref[pl.ds(my * M_BLK, M_BLK), :].astype(jnp.bfloat16)

        @pl.when(t < N_DEV - 1)
        def _prefetch():
            pltpu.make_async_copy(
                w_hbm.at[pl.ds(s_nxt * M_BLK, M_BLK), :],
                w_buf.at[(t + 1) % 2],
                w_sems.at[(t + 1) % 2],
            ).start()

        @pl.when(t != 0)
        def _wait_recv():
            recv = pltpu.make_async_remote_copy(
                src_ref=comm_ref.at[s_cur],
                dst_ref=comm_ref.at[s_cur],
                send_sem=send_sems.at[0],
                recv_sem=recv_sems.at[s_cur],
                device_id=(0,),
                device_id_type=pl.DeviceIdType.MESH,
            )
            recv.wait_recv()

        pltpu.make_async_copy(
            w_hbm.at[pl.ds(s_cur * M_BLK, M_BLK), :],
            w_buf.at[t % 2],
            w_sems.at[t % 2],
        ).wait()

        xblk = comm_ref[s_cur]
        c = 0.7978845608028654
        for ci in range(N_CHUNK):
            seg = pl.ds(ci * N_SEG, N_SEG)
            contrib = jnp.dot(
                xblk,
                w_buf[t % 2, :, seg].astype(jnp.bfloat16),
                preferred_element_type=jnp.float32,
            )

            @pl.when(t == 0)
            def _init(seg=seg, contrib=contrib):
                out_ref[:, seg] = contrib

            @pl.when(jnp.logical_and(t > 0, t < N_DEV - 1))
            def _acc(seg=seg, contrib=contrib):
                out_ref[:, seg] += contrib

            @pl.when(t == N_DEV - 1)
            def _fin(seg=seg, contrib=contrib):
                y = out_ref[:, seg] + contrib
                out_ref[:, seg] = 0.5 * y * (
                    1.0 + jnp.tanh(c * (y + 0.044715 * y * y * y))
                )

        @pl.when(t == N_DEV - 1)
        def _drain():
            for k, r in enumerate(SEND_ORDER):
                tgt = (my + r) % N_DEV
                rdma = pltpu.make_async_remote_copy(
                    src_ref=send_buf.at[tgt],
                    dst_ref=comm_ref.at[my],
                    send_sem=send_sems.at[k],
                    recv_sem=recv_sems.at[my],
                    device_id=(tgt,),
                    device_id_type=pl.DeviceIdType.MESH,
                )
                rdma.wait_send()

    return pl.pallas_call(
        body,
        grid=(N_DEV,),
        out_shape=jax.ShapeDtypeStruct((M_BLK, N_OUT), jnp.float32),
        in_specs=[
            pl.BlockSpec((m_glob, k_shard), lambda t: (0, 0)),
            pl.BlockSpec(memory_space=pl.ANY),
        ],
        out_specs=pl.BlockSpec((M_BLK, N_OUT), lambda t: (0, 0)),
        scratch_shapes=[
            pltpu.VMEM((N_DEV, M_BLK, M_BLK), jnp.bfloat16),
            pltpu.VMEM((N_DEV, M_BLK, M_BLK), jnp.bfloat16),
            pltpu.VMEM((2, M_BLK, N_OUT), jnp.float32),
            pltpu.SemaphoreType.DMA((len(SEND_ORDER),)),
            pltpu.SemaphoreType.DMA((N_DEV,)),
            pltpu.SemaphoreType.DMA((2,)),
        ],
        compiler_params=pltpu.CompilerParams(
            collective_id=0, vmem_limit_bytes=100 * 1024 * 1024
        ),
    )(x, w_mat)


# device time: 131315 ns/iter; 1.2137x vs baseline; 1.2137x over previous
import jax
import jax.numpy as jnp
from jax import lax
from jax.experimental import pallas as pl
from jax.experimental.pallas import tpu as pltpu

N_DEV = 16
M_BLK = 512
N_OUT = 4096
LOOKAHEAD = 4


def kernel(x, w_mat):
    m_glob, k_shard = x.shape
    assert m_glob == N_DEV * M_BLK and k_shard == M_BLK

    def body(
        x_ref, w_hbm, out_ref, send_buf, comm_ref, w_buf, send_sems, recv_sems, w_sems
    ):
        t = pl.program_id(0)
        my = lax.axis_index("i")
        s_cur = (my - t) % N_DEV
        s_nxt = (my - t - 1) % N_DEV

        def send_round(k):
            tgt = (my + k) % N_DEV
            rdma = pltpu.make_async_remote_copy(
                src_ref=send_buf.at[tgt],
                dst_ref=comm_ref.at[my],
                send_sem=send_sems.at[k],
                recv_sem=recv_sems.at[my],
                device_id=(tgt,),
                device_id_type=pl.DeviceIdType.MESH,
            )
            rdma.start()

        @pl.when(t == 0)
        def _setup():
            pltpu.make_async_copy(
                w_hbm.at[pl.ds(s_cur * M_BLK, M_BLK), :], w_buf.at[0], w_sems.at[0]
            ).start()

            for j in range(N_DEV):
                send_buf[j] = x_ref[pl.ds(j * M_BLK, M_BLK), :].astype(
                    jnp.bfloat16
                )
            comm_ref[my] = send_buf[my]

            barrier = pltpu.get_barrier_semaphore()
            for d in range(N_DEV):
                pl.semaphore_signal(
                    barrier,
                    inc=1,
                    device_id=(d,),
                    device_id_type=pl.DeviceIdType.MESH,
                )
            pl.semaphore_wait(barrier, N_DEV)

            for k in range(1, 1 + LOOKAHEAD):
                send_round(k)

        @pl.when(jnp.logical_and(t > 0, t + LOOKAHEAD < N_DEV))
        def _send_ahead():
            send_round(t + LOOKAHEAD)

        @pl.when(t < N_DEV - 1)
        def _prefetch():
            pltpu.make_async_copy(
                w_hbm.at[pl.ds(s_nxt * M_BLK, M_BLK), :],
                w_buf.at[(t + 1) % 2],
                w_sems.at[(t + 1) % 2],
            ).start()

        @pl.when(t != 0)
        def _wait_recv():
            recv = pltpu.make_async_remote_copy(
                src_ref=comm_ref.at[s_cur],
                dst_ref=comm_ref.at[s_cur],
                send_sem=send_sems.at[0],
                recv_sem=recv_sems.at[s_cur],
                device_id=(0,),
                device_id_type=pl.DeviceIdType.MESH,
            )
            recv.wait_recv()

        pltpu.make_async_copy(
            w_hbm.at[pl.ds(s_cur * M_BLK, M_BLK), :],
            w_buf.at[t % 2],
            w_sems.at[t % 2],
        ).wait()

        contrib = jnp.dot(
            comm_ref[s_cur],
            w_buf[t % 2].astype(jnp.bfloat16),
            preferred_element_type=jnp.float32,
        )

        @pl.when(t == 0)
        def _init():
            out_ref[...] = contrib

        @pl.when(jnp.logical_and(t > 0, t < N_DEV - 1))
        def _acc():
            out_ref[...] += contrib

        @pl.when(t == N_DEV - 1)
        def _fin():
            y = out_ref[...] + contrib
            c = 0.7978845608028654
            out_ref[...] = 0.5 * y * (1.0 + jnp.tanh(c * (y + 0.044715 * y * y * y)))

            for k in range(1, N_DEV):
                tgt = (my + k) % N_DEV
                rdma = pltpu.make_async_remote_copy(
                    src_ref=send_buf.at[tgt],
                    dst_ref=comm_ref.at[my],
                    send_sem=send_sems.at[k],
                    recv_sem=recv_sems.at[my],
                    device_id=(tgt,),
                    device_id_type=pl.DeviceIdType.MESH,
                )
                rdma.wait_send()

    return pl.pallas_call(
        body,
        grid=(N_DEV,),
        out_shape=jax.ShapeDtypeStruct((M_BLK, N_OUT), jnp.float32),
        in_specs=[
            pl.BlockSpec((m_glob, k_shard), lambda t: (0, 0)),
            pl.BlockSpec(memory_space=pl.ANY),
        ],
        out_specs=pl.BlockSpec((M_BLK, N_OUT), lambda t: (0, 0)),
        scratch_shapes=[
            pltpu.VMEM((N_DEV, M_BLK, M_BLK), jnp.bfloat16),
            pltpu.VMEM((N_DEV, M_BLK, M_BLK), jnp.bfloat16),
            pltpu.VMEM((2, M_BLK, N_OUT), jnp.float32),
            pltpu.SemaphoreType.DMA((N_DEV,)),
            pltpu.SemaphoreType.DMA((N_DEV,)),
            pltpu.SemaphoreType.DMA((2,)),
        ],
        compiler_params=pltpu.CompilerParams(
            collective_id=0, vmem_limit_bytes=100 * 1024 * 1024
        ),
    )(x, w_mat)
